# baseline (speedup 1.0000x reference)
import functools

import jax
import jax.numpy as jnp
from jax import lax
from jax.experimental import pallas as pl
from jax.experimental.pallas import tpu as pltpu
from jax.experimental.pallas import tpu_sc as plsc

N_USERS = 100000
N_ITEMS = 100000
DIM = 128
BATCH = 16384

NC = 2
NS = 16
L = 16
NW = NC * NS
BPW = BATCH // NW
CHUNK = 128
NCHUNK = BPW // CHUNK


def _gmf_body(user_hbm, item_hbm, utab_hbm, itab_hbm, out_hbm,
              u_idx, i_idx, u_rows, i_rows, o_v, sem_u, sem_i):
    wid = lax.axis_index("s") * NC + lax.axis_index("c")

    pltpu.sync_copy(user_hbm.at[wid], u_idx)
    pltpu.sync_copy(item_hbm.at[wid], i_idx)

    iota = lax.broadcasted_iota(jnp.int32, (L,), 0)

    for c in range(NCHUNK):
        cu = pltpu.async_copy(utab_hbm.at[u_idx.at[pl.ds(c * CHUNK, CHUNK)]],
                              u_rows, sem_u)
        ci = pltpu.async_copy(itab_hbm.at[i_idx.at[pl.ds(c * CHUNK, CHUNK)]],
                              i_rows, sem_i)
        cu.wait()
        ci.wait()

        for g in range(CHUNK // L):
            rvec = iota + (g * L)

            def dot_step(k, acc, rvec=rvec):
                for j in range(8):
                    d = k * 8 + j
                    dvec = jnp.zeros((L,), jnp.int32) + d
                    uv = plsc.load_gather(u_rows, [rvec, dvec])
                    iv = plsc.load_gather(i_rows, [rvec, dvec])
                    acc = acc + uv * iv
                return acc

            acc = lax.fori_loop(0, DIM // 8, dot_step,
                                jnp.zeros((L,), jnp.float32))
            o_v[pl.ds(c * CHUNK + g * L, L)] = 1.0 / (1.0 + jnp.exp(-acc))

    pltpu.sync_copy(o_v, out_hbm.at[wid])


@jax.jit
def _gmf(user2d, item2d, user_table, item_table):
    mesh = plsc.VectorSubcoreMesh(core_axis_name="c", subcore_axis_name="s")
    kern = pl.kernel(
        _gmf_body,
        mesh=mesh,
        out_type=jax.ShapeDtypeStruct((NW, BPW), jnp.float32),
        compiler_params=pltpu.CompilerParams(needs_layout_passes=False),
        scratch_types=[
            pltpu.VMEM((BPW,), jnp.int32),
            pltpu.VMEM((BPW,), jnp.int32),
            pltpu.VMEM((CHUNK, DIM), jnp.float32),
            pltpu.VMEM((CHUNK, DIM), jnp.float32),
            pltpu.VMEM((BPW,), jnp.float32),
            pltpu.SemaphoreType.DMA,
            pltpu.SemaphoreType.DMA,
        ],
    )
    return kern(user2d, item2d, user_table, item_table)


def kernel(user, item, user_table, item_table):
    user2d = user.astype(jnp.int32).reshape(NW, BPW)
    item2d = item.astype(jnp.int32).reshape(NW, BPW)
    out = _gmf(user2d, item2d, user_table, item_table)
    return out.reshape(BATCH)


# speedup vs baseline: 2.3645x; 2.3645x over previous
import jax
import jax.numpy as jnp
from jax import lax
from jax.experimental import pallas as pl
from jax.experimental.pallas import tpu as pltpu
from jax.experimental.pallas import tpu_sc as plsc

DIM = 128
BATCH = 16384

NC = 2
NS = 16
L = 16
NW = NC * NS
BPW = BATCH // NW
CHUNK = 128
NCHUNK = BPW // CHUNK
GROUPS = CHUNK // L


def _gmf_body(user_hbm, item_hbm, utab_hbm, itab_hbm, out_hbm,
              u_idx, i_idx, u_rows0, i_rows0, u_rows1, i_rows1, o_v, accs,
              sem_u0, sem_i0, sem_u1, sem_i1):
    wid = lax.axis_index("s") * NC + lax.axis_index("c")

    pltpu.sync_copy(user_hbm.at[wid], u_idx)
    pltpu.sync_copy(item_hbm.at[wid], i_idx)

    iota = lax.broadcasted_iota(jnp.int32, (L,), 0)

    bufs = [(u_rows0, i_rows0, sem_u0, sem_i0),
            (u_rows1, i_rows1, sem_u1, sem_i1)]

    def issue(c):
        ub, ib, su, si = bufs[c % 2]
        cu = pltpu.async_copy(utab_hbm.at[u_idx.at[pl.ds(c * CHUNK, CHUNK)]],
                              ub, su)
        ci = pltpu.async_copy(itab_hbm.at[i_idx.at[pl.ds(c * CHUNK, CHUNK)]],
                              ib, si)
        return cu, ci

    inflight = issue(0)
    for c in range(NCHUNK):
        if c + 1 < NCHUNK:
            nxt = issue(c + 1)
        inflight[0].wait()
        inflight[1].wait()
        ub, ib, _, _ = bufs[c % 2]

        def group_body(g, _, ub=ub, ib=ib, c=c):
            for rr in range(L):
                r = g * L + rr
                p0 = ub[r, pl.ds(0 * L, L)] * ib[r, pl.ds(0 * L, L)]
                p1 = ub[r, pl.ds(1 * L, L)] * ib[r, pl.ds(1 * L, L)]
                p2 = ub[r, pl.ds(2 * L, L)] * ib[r, pl.ds(2 * L, L)]
                p3 = ub[r, pl.ds(3 * L, L)] * ib[r, pl.ds(3 * L, L)]
                p4 = ub[r, pl.ds(4 * L, L)] * ib[r, pl.ds(4 * L, L)]
                p5 = ub[r, pl.ds(5 * L, L)] * ib[r, pl.ds(5 * L, L)]
                p6 = ub[r, pl.ds(6 * L, L)] * ib[r, pl.ds(6 * L, L)]
                p7 = ub[r, pl.ds(7 * L, L)] * ib[r, pl.ds(7 * L, L)]
                s = ((p0 + p1) + (p2 + p3)) + ((p4 + p5) + (p6 + p7))
                accs[rr, pl.ds(0, L)] = s
            tot = plsc.load_gather(accs, [iota, jnp.zeros((L,), jnp.int32)])
            for j in range(1, L):
                col = plsc.load_gather(
                    accs, [iota, jnp.zeros((L,), jnp.int32) + j])
                tot = tot + col
            o_v[pl.ds(c * CHUNK + g * L, L)] = 1.0 / (1.0 + jnp.exp(-tot))
            return 0

        lax.fori_loop(0, GROUPS, group_body, 0)
        inflight = nxt if c + 1 < NCHUNK else inflight

    pltpu.sync_copy(o_v, out_hbm.at[wid])


@jax.jit
def _gmf(user2d, item2d, user_table, item_table):
    mesh = plsc.VectorSubcoreMesh(core_axis_name="c", subcore_axis_name="s")
    kern = pl.kernel(
        _gmf_body,
        mesh=mesh,
        out_type=jax.ShapeDtypeStruct((NW, BPW), jnp.float32),
        compiler_params=pltpu.CompilerParams(needs_layout_passes=False),
        scratch_types=[
            pltpu.VMEM((BPW,), jnp.int32),
            pltpu.VMEM((BPW,), jnp.int32),
            pltpu.VMEM((CHUNK, DIM), jnp.float32),
            pltpu.VMEM((CHUNK, DIM), jnp.float32),
            pltpu.VMEM((CHUNK, DIM), jnp.float32),
            pltpu.VMEM((CHUNK, DIM), jnp.float32),
            pltpu.VMEM((BPW,), jnp.float32),
            pltpu.VMEM((L, L), jnp.float32),
            pltpu.SemaphoreType.DMA,
            pltpu.SemaphoreType.DMA,
            pltpu.SemaphoreType.DMA,
            pltpu.SemaphoreType.DMA,
        ],
    )
    return kern(user2d, item2d, user_table, item_table)


def kernel(user, item, user_table, item_table):
    user2d = user.astype(jnp.int32).reshape(NW, BPW)
    item2d = item.astype(jnp.int32).reshape(NW, BPW)
    out = _gmf(user2d, item2d, user_table, item_table)
    return out.reshape(BATCH)


# speedup vs baseline: 2.4416x; 1.0326x over previous
import jax
import jax.numpy as jnp
from jax import lax
from jax.experimental import pallas as pl
from jax.experimental.pallas import tpu as pltpu
from jax.experimental.pallas import tpu_sc as plsc

DIM = 128
BATCH = 16384

NC = 2
NS = 16
L = 16
NW = NC * NS
BPW = BATCH // NW
CHUNK = 128
NCHUNK = BPW // CHUNK
GROUPS = CHUNK // L


def _gmf_body(user_hbm, item_hbm, utab_hbm, itab_hbm, out_hbm,
              u_idx, i_idx, u_rows0, i_rows0, u_rows1, i_rows1, o_v, accs,
              sem_u0, sem_i0, sem_u1, sem_i1):
    wid = lax.axis_index("s") * NC + lax.axis_index("c")
    base = wid * BPW

    pltpu.sync_copy(user_hbm.at[pl.ds(base, BPW)], u_idx)
    pltpu.sync_copy(item_hbm.at[pl.ds(base, BPW)], i_idx)

    iota = lax.broadcasted_iota(jnp.int32, (L,), 0)

    bufs = [(u_rows0, i_rows0, sem_u0, sem_i0),
            (u_rows1, i_rows1, sem_u1, sem_i1)]

    def issue(c):
        ub, ib, su, si = bufs[c % 2]
        cu = pltpu.async_copy(utab_hbm.at[u_idx.at[pl.ds(c * CHUNK, CHUNK)]],
                              ub, su)
        ci = pltpu.async_copy(itab_hbm.at[i_idx.at[pl.ds(c * CHUNK, CHUNK)]],
                              ib, si)
        return cu, ci

    inflight = issue(0)
    for c in range(NCHUNK):
        if c + 1 < NCHUNK:
            nxt = issue(c + 1)
        inflight[0].wait()
        inflight[1].wait()
        ub, ib, _, _ = bufs[c % 2]

        def group_body(g, _, ub=ub, ib=ib, c=c):
            for rr in range(L):
                r = g * L + rr
                p0 = ub[r, pl.ds(0 * L, L)] * ib[r, pl.ds(0 * L, L)]
                p1 = ub[r, pl.ds(1 * L, L)] * ib[r, pl.ds(1 * L, L)]
                p2 = ub[r, pl.ds(2 * L, L)] * ib[r, pl.ds(2 * L, L)]
                p3 = ub[r, pl.ds(3 * L, L)] * ib[r, pl.ds(3 * L, L)]
                p4 = ub[r, pl.ds(4 * L, L)] * ib[r, pl.ds(4 * L, L)]
                p5 = ub[r, pl.ds(5 * L, L)] * ib[r, pl.ds(5 * L, L)]
                p6 = ub[r, pl.ds(6 * L, L)] * ib[r, pl.ds(6 * L, L)]
                p7 = ub[r, pl.ds(7 * L, L)] * ib[r, pl.ds(7 * L, L)]
                s = ((p0 + p1) + (p2 + p3)) + ((p4 + p5) + (p6 + p7))
                accs[rr, pl.ds(0, L)] = s
            tot = plsc.load_gather(accs, [iota, jnp.zeros((L,), jnp.int32)])
            for j in range(1, L):
                col = plsc.load_gather(
                    accs, [iota, jnp.zeros((L,), jnp.int32) + j])
                tot = tot + col
            o_v[pl.ds(c * CHUNK + g * L, L)] = 1.0 / (1.0 + jnp.exp(-tot))
            return 0

        lax.fori_loop(0, GROUPS, group_body, 0)
        inflight = nxt if c + 1 < NCHUNK else inflight

    pltpu.sync_copy(o_v, out_hbm.at[pl.ds(base, BPW)])


@jax.jit
def _gmf(user1d, item1d, user_table, item_table):
    mesh = plsc.VectorSubcoreMesh(core_axis_name="c", subcore_axis_name="s")
    kern = pl.kernel(
        _gmf_body,
        mesh=mesh,
        out_type=jax.ShapeDtypeStruct((BATCH,), jnp.float32),
        compiler_params=pltpu.CompilerParams(needs_layout_passes=False),
        scratch_types=[
            pltpu.VMEM((BPW,), jnp.int32),
            pltpu.VMEM((BPW,), jnp.int32),
            pltpu.VMEM((CHUNK, DIM), jnp.float32),
            pltpu.VMEM((CHUNK, DIM), jnp.float32),
            pltpu.VMEM((CHUNK, DIM), jnp.float32),
            pltpu.VMEM((CHUNK, DIM), jnp.float32),
            pltpu.VMEM((BPW,), jnp.float32),
            pltpu.VMEM((L, L), jnp.float32),
            pltpu.SemaphoreType.DMA,
            pltpu.SemaphoreType.DMA,
            pltpu.SemaphoreType.DMA,
            pltpu.SemaphoreType.DMA,
        ],
    )
    return kern(user1d, item1d, user_table, item_table)


def kernel(user, item, user_table, item_table):
    return _gmf(user.astype(jnp.int32), item.astype(jnp.int32),
                user_table, item_table)
